# baseline (speedup 1.0000x reference)
import functools

import jax
import jax.numpy as jnp
from jax import lax
from jax.experimental import pallas as pl
from jax.experimental.pallas import tpu as pltpu
from jax.experimental.pallas import tpu_sc as plsc

_B = 4096
_BB = 128
_NW = 32
_CHUNK = _B // _NW

_F32 = jnp.float32
_BF16 = jnp.bfloat16


def _sc_gather(idx_all, idx_ud, tab_all, tab_ud):
    mesh = plsc.VectorSubcoreMesh(core_axis_name="c", subcore_axis_name="s")

    @functools.partial(
        pl.kernel,
        mesh=mesh,
        out_type=(jax.ShapeDtypeStruct((_B * 13, 128), _F32),
                  jax.ShapeDtypeStruct((_B, 128), _F32)),
        scratch_types=[
            pltpu.VMEM((_CHUNK,), jnp.int32),
            pltpu.VMEM((_CHUNK, 128), _F32),
            pltpu.SemaphoreType.DMA,
        ],
    )
    def k(idx_all_ref, idx_ud_ref, tab_all_ref, tab_ud_ref, out_ref,
          out_ud_ref, idx_v, rows_v, sem):
        wid = lax.axis_index("s") * 2 + lax.axis_index("c")
        base = wid * (13 * _CHUNK)
        for j in range(13):
            off = base + j * _CHUNK
            pltpu.sync_copy(idx_all_ref.at[pl.ds(off, _CHUNK)], idx_v)
            pltpu.async_copy(tab_all_ref.at[idx_v], rows_v, sem).wait()
            pltpu.sync_copy(rows_v, out_ref.at[pl.ds(off, _CHUNK), :])
        ud_base = wid * _CHUNK
        pltpu.sync_copy(idx_ud_ref.at[pl.ds(ud_base, _CHUNK)], idx_v)
        pltpu.async_copy(tab_ud_ref.at[idx_v], rows_v, sem).wait()
        pltpu.sync_copy(rows_v, out_ud_ref.at[pl.ds(ud_base, _CHUNK), :])

    return k(idx_all, idx_ud, tab_all, tab_ud)


def _tc_body(vat_ref, x2_ref, cat_ref, ud_ref, kcat_ref, appb_ref, ev_ref,
             kv_ref, vb_ref, pW1_ref, pb1_ref, pW2_ref, pb2_ref, uW1c_ref,
             uW1ud_ref, uWa_ref, uWv_ref, ub1_ref, uW2_ref, ub2_ref,
             rW1_ref, rb1_ref, rW2_ref, rb2_ref, out_ref):
    _up = lambda a, t: pltpu.roll(a, a.shape[0] - t, 0)

    x = x2_ref[...]
    P = jnp.dot(x.astype(_BF16), kcat_ref[...],
                preferred_element_type=_F32).astype(_BF16)
    y = (P[:, 0:32] + _up(P, 1)[:, 32:64] + _up(P, 2)[:, 64:96]
         + appb_ref[...])

    vat = vat_ref[...]
    cls = lax.broadcasted_iota(jnp.int32, (1, 3), 1)
    oh = jnp.where(vat == cls, 1.0, 0.0).astype(_BF16)
    ev = ev_ref[...]
    mvit = jnp.concatenate(
        [jnp.dot(ev, kv_ref[t], preferred_element_type=_F32) for t in range(3)],
        axis=1).astype(_BF16)
    Pv = jnp.dot(oh, mvit, preferred_element_type=_F32).astype(_BF16)
    yv = (Pv[:, 0:32] + _up(Pv, 1)[:, 32:64] + _up(Pv, 2)[:, 64:96]
          + vb_ref[...])

    ya3 = jnp.maximum(y, _BF16(0.0)).reshape(_BB, 30, 32)
    za3 = jnp.maximum(ya3[:, 0:29, :], ya3[:, 1:30, :])
    app = za3.reshape(_BB, 928)
    yv3 = jnp.maximum(yv, _BF16(0.0)).reshape(_BB, 30, 32)
    zv3 = jnp.maximum(yv3[:, 0:29, :], yv3[:, 1:30, :])
    vit = zv3.reshape(_BB, 928)

    cat = cat_ref[...]
    u1 = (jnp.dot(cat, uW1c_ref[...], preferred_element_type=_F32)
          + jnp.dot(ud_ref[...], uW1ud_ref[...], preferred_element_type=_F32)
          + jnp.dot(app, uWa_ref[...], preferred_element_type=_F32)
          + jnp.dot(vit, uWv_ref[...], preferred_element_type=_F32)
          + ub1_ref[...])
    u1 = jnp.maximum(u1, 0.0)
    ue = jnp.maximum(
        jnp.dot(u1, uW2_ref[...], preferred_element_type=_F32) + ub2_ref[...],
        0.0)
    p1 = jnp.maximum(
        jnp.dot(cat, pW1_ref[...], preferred_element_type=_F32) + pb1_ref[...],
        0.0)
    pe = jnp.maximum(
        jnp.dot(p1, pW2_ref[...], preferred_element_type=_F32) + pb2_ref[...],
        0.0)
    h = jnp.concatenate([ue, pe], axis=1)
    r1 = jnp.maximum(
        jnp.dot(h, rW1_ref[...], preferred_element_type=_F32) + rb1_ref[...],
        0.0)
    o = jnp.dot(r1, rW2_ref[...], preferred_element_type=_F32) + rb2_ref[...]
    out_ref[...] = jax.nn.sigmoid(o)


def _tc_towers(vat, x2, cat, ud, kcat, appb, ev, kv, vb, pW1, pb1, pW2, pb2,
               uW1c, uW1ud, uWa, uWv, ub1, uW2, ub2, rW1, rb1, rW2, rb2):
    def full(a):
        nd = a.ndim
        return pl.BlockSpec(a.shape, lambda i, _n=nd: (0,) * _n)

    weights = (kcat, appb, ev, kv, vb, pW1, pb1, pW2, pb2,
               uW1c, uW1ud, uWa, uWv, ub1, uW2, ub2, rW1, rb1, rW2, rb2)
    return pl.pallas_call(
        _tc_body,
        grid=(_B // _BB,),
        in_specs=[
            pl.BlockSpec((_BB * 30, 1), lambda i: (i, 0)),
            pl.BlockSpec((_BB * 30, 609), lambda i: (i, 0)),
            pl.BlockSpec((_BB, 13 * 128), lambda i: (i, 0)),
            pl.BlockSpec((_BB, 128), lambda i: (i, 0)),
        ] + [full(w) for w in weights],
        out_specs=pl.BlockSpec((_BB, 1), lambda i: (i, 0)),
        out_shape=jax.ShapeDtypeStruct((_B, 1), _F32),
    )(vat, x2, cat, ud, *weights)


def kernel(brand, modelname, version, phone_log_model, phone_raw_model,
           total_use_days, user_age, user_sex, user_degree, resident_province,
           resident_city, resident_city_type, sale_channel_1, sale_channel_2,
           vatality, all_app, emb_brand, emb_model, emb_version, emb_phone_log,
           emb_phone_raw, phone_W1, phone_b1, phone_W2, phone_b2, emb_age,
           emb_sex, emb_degree, emb_province, emb_city, emb_city_type, emb_ch1,
           emb_ch2, emb_use_days, emb_vitality, conv_vit_k, conv_vit_b,
           conv_app_k, conv_app_b, user_W1, user_b1, user_W2, user_b2,
           rate_W1, rate_b1, rate_W2, rate_b2):
    idx_feats = (brand, modelname, version, phone_log_model, phone_raw_model,
                 user_age, user_sex, user_degree, resident_province,
                 resident_city, resident_city_type, sale_channel_1,
                 sale_channel_2)
    tabs = (emb_brand, emb_model, emb_version, emb_phone_log, emb_phone_raw,
            emb_age, emb_sex, emb_degree, emb_province, emb_city,
            emb_city_type, emb_ch1, emb_ch2)
    tab_all = jnp.concatenate([jnp.pad(t, ((0, 0), (0, 118))) for t in tabs],
                              axis=0)
    offs, acc = [], 0
    for t in tabs:
        offs.append(acc)
        acc += t.shape[0]
    idx_all = jnp.stack(
        [a.reshape(-1).astype(jnp.int32) + o for a, o in zip(idx_feats, offs)],
        axis=1).reshape(-1)
    idx_ud = total_use_days.reshape(-1).astype(jnp.int32)
    tab_ud = jnp.pad(emb_use_days, ((0, 0), (0, 28)))
    out_all, ud = _sc_gather(idx_all, idx_ud, tab_all, tab_ud)
    cat = out_all.reshape(_B, 13 * 128)

    z128 = jnp.zeros((128, 128), _F32)
    z118 = jnp.zeros((118, 128), _F32)
    pw, uw = [], []
    for f in range(5):
        pw += [phone_W1[10 * f:10 * f + 10], z118]
        uw.append(z128)
    for g in range(8):
        pw.append(z128)
        uw += [user_W1[10 * g:10 * g + 10], z118]
    pW1 = jnp.concatenate(pw, axis=0)
    uW1c = jnp.concatenate(uw, axis=0)
    uW1ud = jnp.concatenate([user_W1[80:180], jnp.zeros((28, 128), _F32)],
                            axis=0)
    def pad_wc(w):
        return jnp.pad(w.reshape(27, 32, 128),
                       ((0, 2), (0, 0), (0, 0))).reshape(928, 128).astype(_BF16)

    uWa = pad_wc(user_W1[180:1044])
    uWv = pad_wc(user_W1[1044:1908])
    kcat = jnp.concatenate([conv_app_k[0], conv_app_k[1], conv_app_k[2]],
                           axis=1).astype(_BF16)

    x2 = all_app.reshape(_B * 30, 609)
    return _tc_towers(
        vatality.astype(jnp.int32).reshape(_B * 30, 1), x2, cat, ud,
        kcat, conv_app_b.reshape(1, 32).astype(_BF16), emb_vitality,
        conv_vit_k, conv_vit_b.reshape(1, 32).astype(_BF16),
        pW1, phone_b1.reshape(1, 128), phone_W2, phone_b2.reshape(1, 32),
        uW1c, uW1ud, uWa, uWv, user_b1.reshape(1, 128), user_W2,
        user_b2.reshape(1, 32),
        rate_W1, rate_b1.reshape(1, 64), rate_W2, rate_b2.reshape(1, 1))


# speedup vs baseline: 1.0995x; 1.0995x over previous
import functools

import jax
import jax.numpy as jnp
from jax import lax
from jax.experimental import pallas as pl
from jax.experimental.pallas import tpu as pltpu
from jax.experimental.pallas import tpu_sc as plsc

_B = 4096
_BB = 128
_NW = 32
_CHUNK = _B // _NW

_F32 = jnp.float32
_BF16 = jnp.bfloat16


def _sc_gather(idx_all, idx_ud, tab_all, tab_ud):
    mesh = plsc.VectorSubcoreMesh(core_axis_name="c", subcore_axis_name="s")

    @functools.partial(
        pl.kernel,
        mesh=mesh,
        out_type=(jax.ShapeDtypeStruct((_B * 13, 128), _F32),
                  jax.ShapeDtypeStruct((_B, 128), _F32)),
        scratch_types=[
            pltpu.VMEM((13 * _CHUNK,), jnp.int32),
            pltpu.VMEM((_CHUNK,), jnp.int32),
            pltpu.VMEM((_CHUNK, 128), _F32),
            pltpu.VMEM((_CHUNK, 128), _F32),
            pltpu.SemaphoreType.DMA,
            pltpu.SemaphoreType.DMA,
        ],
    )
    def k(idx_all_ref, idx_ud_ref, tab_all_ref, tab_ud_ref, out_ref,
          out_ud_ref, idx_v, idx_ud_v, buf_a, buf_b, sem_a, sem_b):
        wid = lax.axis_index("s") * 2 + lax.axis_index("c")
        base = wid * (13 * _CHUNK)
        ud_base = wid * _CHUNK
        pltpu.sync_copy(idx_all_ref.at[pl.ds(base, 13 * _CHUNK)], idx_v)
        pltpu.sync_copy(idx_ud_ref.at[pl.ds(ud_base, _CHUNK)], idx_ud_v)
        bufs = (buf_a, buf_b)
        sems = (sem_a, sem_b)
        prev = None
        for j in range(14):
            b = j % 2
            if j < 13:
                cp = pltpu.async_copy(
                    tab_all_ref.at[idx_v.at[pl.ds(_CHUNK * j, _CHUNK)]],
                    bufs[b], sems[b])
            else:
                cp = pltpu.async_copy(tab_ud_ref.at[idx_ud_v], bufs[b], sems[b])
            if prev is not None:
                pj, pcp = prev
                pcp.wait()
                pltpu.sync_copy(
                    bufs[pj % 2],
                    out_ref.at[pl.ds(base + _CHUNK * pj, _CHUNK), :])
            prev = (j, cp)
        prev[1].wait()
        pltpu.sync_copy(bufs[13 % 2], out_ud_ref.at[pl.ds(ud_base, _CHUNK), :])

    return k(idx_all, idx_ud, tab_all, tab_ud)


_TAB_SIZES = (100, 5000, 50, 10000, 10000, 100, 3, 10, 35, 400, 6, 50, 200)


def _prep_body(*refs):
    tab_refs = refs[0:13]
    ud_ref = refs[13]
    out_ref = refs[14]
    out_ud_ref = refs[15]
    off = 0
    for t_ref, v in zip(tab_refs, _TAB_SIZES):
        out_ref[pl.ds(off, v), :] = jnp.concatenate(
            [t_ref[...], jnp.zeros((v, 118), _F32)], axis=1)
        off += v
    out_ud_ref[...] = jnp.concatenate(
        [ud_ref[...], jnp.zeros((5000, 28), _F32)], axis=1)


def _prep_tables(tabs, emb_use_days):
    def full(a):
        nd = a.ndim
        return pl.BlockSpec(a.shape, lambda _n=nd: (0,) * _n)

    return pl.pallas_call(
        _prep_body,
        in_specs=[full(t) for t in tabs] + [full(emb_use_days)],
        out_specs=(pl.BlockSpec((25954, 128), lambda: (0, 0)),
                   pl.BlockSpec((5000, 128), lambda: (0, 0))),
        out_shape=(jax.ShapeDtypeStruct((25954, 128), _F32),
                   jax.ShapeDtypeStruct((5000, 128), _F32)),
    )(*tabs, emb_use_days)


def _tc_body(vat_ref, x2_ref, cat_ref, ud_ref, kcat_ref, appb_ref, ev_ref,
             kv_ref, vb_ref, pW1_ref, pb1_ref, pW2_ref, pb2_ref, uW1c_ref,
             uW1ud_ref, uWa_ref, uWv_ref, ub1_ref, uW2_ref, ub2_ref,
             rW1_ref, rb1_ref, rW2_ref, rb2_ref, out_ref):
    _up = lambda a, t: pltpu.roll(a, a.shape[0] - t, 0)

    x = x2_ref[...]
    P = jnp.dot(x.astype(_BF16), kcat_ref[...],
                preferred_element_type=_F32)
    y = (P[:, 0:32] + _up(P, 1)[:, 32:64] + _up(P, 2)[:, 64:96]
         + appb_ref[...])

    vat = vat_ref[...]
    cls = lax.broadcasted_iota(jnp.int32, (1, 3), 1)
    oh = jnp.where(vat == cls, 1.0, 0.0).astype(_BF16)
    ev = ev_ref[...]
    mvit = jnp.concatenate(
        [jnp.dot(ev, kv_ref[t], preferred_element_type=_F32) for t in range(3)],
        axis=1).astype(_BF16)
    Pv = jnp.dot(oh, mvit, preferred_element_type=_F32)
    yv = (Pv[:, 0:32] + _up(Pv, 1)[:, 32:64] + _up(Pv, 2)[:, 64:96]
          + vb_ref[...])

    ya3 = jnp.maximum(y, 0.0).reshape(_BB, 30, 32)
    za3 = jnp.maximum(ya3[:, 0:29, :], ya3[:, 1:30, :])
    app = za3.reshape(_BB, 928).astype(_BF16)
    yv3 = jnp.maximum(yv, 0.0).reshape(_BB, 30, 32)
    zv3 = jnp.maximum(yv3[:, 0:29, :], yv3[:, 1:30, :])
    vit = zv3.reshape(_BB, 928).astype(_BF16)

    cat = cat_ref[...].astype(_BF16)
    u1 = (jnp.dot(cat, uW1c_ref[...], preferred_element_type=_F32)
          + jnp.dot(ud_ref[...].astype(_BF16), uW1ud_ref[...],
                    preferred_element_type=_F32)
          + jnp.dot(app, uWa_ref[...], preferred_element_type=_F32)
          + jnp.dot(vit, uWv_ref[...], preferred_element_type=_F32)
          + ub1_ref[...])
    u1 = jnp.maximum(u1, 0.0)
    ue = jnp.maximum(
        jnp.dot(u1, uW2_ref[...], preferred_element_type=_F32) + ub2_ref[...],
        0.0)
    p1 = jnp.maximum(
        jnp.dot(cat, pW1_ref[...], preferred_element_type=_F32) + pb1_ref[...],
        0.0)
    pe = jnp.maximum(
        jnp.dot(p1, pW2_ref[...], preferred_element_type=_F32) + pb2_ref[...],
        0.0)
    h = jnp.concatenate([ue, pe], axis=1)
    r1 = jnp.maximum(
        jnp.dot(h, rW1_ref[...], preferred_element_type=_F32) + rb1_ref[...],
        0.0)
    o = jnp.dot(r1, rW2_ref[...], preferred_element_type=_F32) + rb2_ref[...]
    out_ref[...] = jax.nn.sigmoid(o)


def _tc_towers(vat, x2, cat, ud, kcat, appb, ev, kv, vb, pW1, pb1, pW2, pb2,
               uW1c, uW1ud, uWa, uWv, ub1, uW2, ub2, rW1, rb1, rW2, rb2):
    def full(a):
        nd = a.ndim
        return pl.BlockSpec(a.shape, lambda i, _n=nd: (0,) * _n)

    weights = (kcat, appb, ev, kv, vb, pW1, pb1, pW2, pb2,
               uW1c, uW1ud, uWa, uWv, ub1, uW2, ub2, rW1, rb1, rW2, rb2)
    return pl.pallas_call(
        _tc_body,
        grid=(_B // _BB,),
        in_specs=[
            pl.BlockSpec((_BB * 30, 1), lambda i: (i, 0)),
            pl.BlockSpec((_BB * 30, 609), lambda i: (i, 0)),
            pl.BlockSpec((_BB, 13 * 128), lambda i: (i, 0)),
            pl.BlockSpec((_BB, 128), lambda i: (i, 0)),
        ] + [full(w) for w in weights],
        out_specs=pl.BlockSpec((_BB, 1), lambda i: (i, 0)),
        out_shape=jax.ShapeDtypeStruct((_B, 1), _F32),
    )(vat, x2, cat, ud, *weights)


def kernel(brand, modelname, version, phone_log_model, phone_raw_model,
           total_use_days, user_age, user_sex, user_degree, resident_province,
           resident_city, resident_city_type, sale_channel_1, sale_channel_2,
           vatality, all_app, emb_brand, emb_model, emb_version, emb_phone_log,
           emb_phone_raw, phone_W1, phone_b1, phone_W2, phone_b2, emb_age,
           emb_sex, emb_degree, emb_province, emb_city, emb_city_type, emb_ch1,
           emb_ch2, emb_use_days, emb_vitality, conv_vit_k, conv_vit_b,
           conv_app_k, conv_app_b, user_W1, user_b1, user_W2, user_b2,
           rate_W1, rate_b1, rate_W2, rate_b2):
    idx_feats = (brand, modelname, version, phone_log_model, phone_raw_model,
                 user_age, user_sex, user_degree, resident_province,
                 resident_city, resident_city_type, sale_channel_1,
                 sale_channel_2)
    tabs = (emb_brand, emb_model, emb_version, emb_phone_log, emb_phone_raw,
            emb_age, emb_sex, emb_degree, emb_province, emb_city,
            emb_city_type, emb_ch1, emb_ch2)
    tab_all, tab_ud = _prep_tables(tabs, emb_use_days)
    offs, acc = [], 0
    for t in tabs:
        offs.append(acc)
        acc += t.shape[0]
    idx_all = jnp.stack(
        [a.reshape(-1).astype(jnp.int32) + o for a, o in zip(idx_feats, offs)],
        axis=1).reshape(-1)
    idx_ud = total_use_days.reshape(-1).astype(jnp.int32)
    out_all, ud = _sc_gather(idx_all, idx_ud, tab_all, tab_ud)
    cat = out_all.reshape(_B, 13 * 128)

    z128 = jnp.zeros((128, 128), _F32)
    z118 = jnp.zeros((118, 128), _F32)
    pw, uw = [], []
    for f in range(5):
        pw += [phone_W1[10 * f:10 * f + 10], z118]
        uw.append(z128)
    for g in range(8):
        pw.append(z128)
        uw += [user_W1[10 * g:10 * g + 10], z118]
    pW1 = jnp.concatenate(pw, axis=0).astype(_BF16)
    uW1c = jnp.concatenate(uw, axis=0).astype(_BF16)
    uW1ud = jnp.concatenate([user_W1[80:180], jnp.zeros((28, 128), _F32)],
                            axis=0).astype(_BF16)
    def pad_wc(w):
        return jnp.pad(w.reshape(27, 32, 128),
                       ((0, 2), (0, 0), (0, 0))).reshape(928, 128).astype(_BF16)

    uWa = pad_wc(user_W1[180:1044])
    uWv = pad_wc(user_W1[1044:1908])
    kcat = jnp.concatenate([conv_app_k[0], conv_app_k[1], conv_app_k[2]],
                           axis=1).astype(_BF16)

    x2 = all_app.reshape(_B * 30, 609)
    return _tc_towers(
        vatality.astype(jnp.int32).reshape(_B * 30, 1), x2, cat, ud,
        kcat, conv_app_b.reshape(1, 32).astype(_BF16), emb_vitality,
        conv_vit_k, conv_vit_b.reshape(1, 32).astype(_BF16),
        pW1, phone_b1.reshape(1, 128), phone_W2, phone_b2.reshape(1, 32),
        uW1c, uW1ud, uWa, uWv, user_b1.reshape(1, 128), user_W2,
        user_b2.reshape(1, 32),
        rate_W1, rate_b1.reshape(1, 64), rate_W2, rate_b2.reshape(1, 1))


# speedup vs baseline: 1.2694x; 1.1545x over previous
import functools

import jax
import jax.numpy as jnp
from jax import lax
from jax.experimental import pallas as pl
from jax.experimental.pallas import tpu as pltpu
from jax.experimental.pallas import tpu_sc as plsc

_B = 4096
_BB = 128
_NW = 32
_CHUNK = _B // _NW

_F32 = jnp.float32
_BF16 = jnp.bfloat16


def _sc_gather(idx_all, idx_ud, tab_all, tab_ud):
    mesh = plsc.VectorSubcoreMesh(core_axis_name="c", subcore_axis_name="s")

    @functools.partial(
        pl.kernel,
        mesh=mesh,
        out_type=(jax.ShapeDtypeStruct((_B * 13, 128), _F32),
                  jax.ShapeDtypeStruct((_B, 128), _F32)),
        scratch_types=[
            pltpu.VMEM((13 * _CHUNK,), jnp.int32),
            pltpu.VMEM((_CHUNK,), jnp.int32),
            pltpu.VMEM((_CHUNK, 128), _F32),
            pltpu.VMEM((_CHUNK, 128), _F32),
            pltpu.SemaphoreType.DMA,
            pltpu.SemaphoreType.DMA,
        ],
    )
    def k(idx_all_ref, idx_ud_ref, tab_all_ref, tab_ud_ref, out_ref,
          out_ud_ref, idx_v, idx_ud_v, buf_a, buf_b, sem_a, sem_b):
        wid = lax.axis_index("s") * 2 + lax.axis_index("c")
        base = wid * (13 * _CHUNK)
        ud_base = wid * _CHUNK
        pltpu.sync_copy(idx_all_ref.at[pl.ds(base, 13 * _CHUNK)], idx_v)
        pltpu.sync_copy(idx_ud_ref.at[pl.ds(ud_base, _CHUNK)], idx_ud_v)
        bufs = (buf_a, buf_b)
        sems = (sem_a, sem_b)
        prev = None
        for j in range(14):
            b = j % 2
            if j < 13:
                cp = pltpu.async_copy(
                    tab_all_ref.at[idx_v.at[pl.ds(_CHUNK * j, _CHUNK)]],
                    bufs[b], sems[b])
            else:
                cp = pltpu.async_copy(tab_ud_ref.at[idx_ud_v], bufs[b], sems[b])
            if prev is not None:
                pj, pcp = prev
                pcp.wait()
                pltpu.sync_copy(
                    bufs[pj % 2],
                    out_ref.at[pl.ds(base + _CHUNK * pj, _CHUNK), :])
            prev = (j, cp)
        prev[1].wait()
        pltpu.sync_copy(bufs[13 % 2], out_ud_ref.at[pl.ds(ud_base, _CHUNK), :])

    return k(idx_all, idx_ud, tab_all, tab_ud)


_TAB_SIZES = (100, 5000, 50, 10000, 10000, 100, 3, 10, 35, 400, 6, 50, 200)


def _prep_body(*refs):
    tab_refs = refs[0:13]
    ud_ref = refs[13]
    out_ref = refs[14]
    out_ud_ref = refs[15]
    off = 0
    for t_ref, v in zip(tab_refs, _TAB_SIZES):
        out_ref[pl.ds(off, v), :] = jnp.concatenate(
            [t_ref[...], jnp.zeros((v, 118), _F32)], axis=1)
        off += v
    out_ud_ref[...] = jnp.concatenate(
        [ud_ref[...], jnp.zeros((5000, 28), _F32)], axis=1)


def _prep_tables(tabs, emb_use_days):
    def full(a):
        nd = a.ndim
        return pl.BlockSpec(a.shape, lambda _n=nd: (0,) * _n)

    return pl.pallas_call(
        _prep_body,
        in_specs=[full(t) for t in tabs] + [full(emb_use_days)],
        out_specs=(pl.BlockSpec((25954, 128), lambda: (0, 0)),
                   pl.BlockSpec((5000, 128), lambda: (0, 0))),
        out_shape=(jax.ShapeDtypeStruct((25954, 128), _F32),
                   jax.ShapeDtypeStruct((5000, 128), _F32)),
    )(*tabs, emb_use_days)


def _tc_body(vat_ref, x2_ref, cat_ref, ud_ref, kcat_ref, appb_ref, wvb_ref,
             vbB_ref, pW1_ref, pb1_ref, pW2_ref, pb2_ref, uW1c_ref,
             uW1ud_ref, uWa_ref, uWv_ref, ub1_ref, uW2_ref, ub2_ref,
             rW1_ref, rb1_ref, rW2_ref, rb2_ref, out_ref):
    _up = lambda a, t: pltpu.roll(a, a.shape[0] - t, 0)

    x = x2_ref[...]
    P = jnp.dot(x.astype(_BF16), kcat_ref[...],
                preferred_element_type=_F32)
    y = (P[:, 0:32] + _up(P, 1)[:, 32:64] + _up(P, 2)[:, 64:96]
         + appb_ref[...])

    vat = vat_ref[...]
    oh = jnp.concatenate(
        [jnp.where(vat == c, 1.0, 0.0) for c in range(3)],
        axis=1).astype(_BF16)
    yvB = jnp.dot(oh, wvb_ref[...],
                  preferred_element_type=_F32) + vbB_ref[...]
    yvB = jnp.maximum(yvB, 0.0)
    vit = jnp.maximum(yvB[:, 0:864], yvB[:, 32:896]).astype(_BF16)

    ya3 = jnp.maximum(y, 0.0).reshape(_BB, 30, 32)
    za3 = jnp.maximum(ya3[:, 0:29, :], ya3[:, 1:30, :])
    app = za3.reshape(_BB, 928).astype(_BF16)

    cat = cat_ref[...].astype(_BF16)
    u1 = (jnp.dot(cat, uW1c_ref[...], preferred_element_type=_F32)
          + jnp.dot(ud_ref[...].astype(_BF16), uW1ud_ref[...],
                    preferred_element_type=_F32)
          + jnp.dot(app, uWa_ref[...], preferred_element_type=_F32)
          + jnp.dot(vit, uWv_ref[...], preferred_element_type=_F32)
          + ub1_ref[...])
    u1 = jnp.maximum(u1, 0.0)
    ue = jnp.maximum(
        jnp.dot(u1, uW2_ref[...], preferred_element_type=_F32) + ub2_ref[...],
        0.0)
    p1 = jnp.maximum(
        jnp.dot(cat, pW1_ref[...], preferred_element_type=_F32) + pb1_ref[...],
        0.0)
    pe = jnp.maximum(
        jnp.dot(p1, pW2_ref[...], preferred_element_type=_F32) + pb2_ref[...],
        0.0)
    h = jnp.concatenate([ue, pe], axis=1)
    r1 = jnp.maximum(
        jnp.dot(h, rW1_ref[...], preferred_element_type=_F32) + rb1_ref[...],
        0.0)
    o = jnp.dot(r1, rW2_ref[...], preferred_element_type=_F32) + rb2_ref[...]
    out_ref[...] = jax.nn.sigmoid(o)


def _tc_towers(vat, x2, cat, ud, kcat, appb, wvb, vbB, pW1, pb1, pW2, pb2,
               uW1c, uW1ud, uWa, uWv, ub1, uW2, ub2, rW1, rb1, rW2, rb2):
    def full(a):
        nd = a.ndim
        return pl.BlockSpec(a.shape, lambda i, _n=nd: (0,) * _n)

    weights = (kcat, appb, wvb, vbB, pW1, pb1, pW2, pb2,
               uW1c, uW1ud, uWa, uWv, ub1, uW2, ub2, rW1, rb1, rW2, rb2)
    return pl.pallas_call(
        _tc_body,
        grid=(_B // _BB,),
        in_specs=[
            pl.BlockSpec((_BB, 30), lambda i: (i, 0)),
            pl.BlockSpec((_BB * 30, 609), lambda i: (i, 0)),
            pl.BlockSpec((_BB, 13 * 128), lambda i: (i, 0)),
            pl.BlockSpec((_BB, 128), lambda i: (i, 0)),
        ] + [full(w) for w in weights],
        out_specs=pl.BlockSpec((_BB, 1), lambda i: (i, 0)),
        out_shape=jax.ShapeDtypeStruct((_B, 1), _F32),
    )(vat, x2, cat, ud, *weights)


def kernel(brand, modelname, version, phone_log_model, phone_raw_model,
           total_use_days, user_age, user_sex, user_degree, resident_province,
           resident_city, resident_city_type, sale_channel_1, sale_channel_2,
           vatality, all_app, emb_brand, emb_model, emb_version, emb_phone_log,
           emb_phone_raw, phone_W1, phone_b1, phone_W2, phone_b2, emb_age,
           emb_sex, emb_degree, emb_province, emb_city, emb_city_type, emb_ch1,
           emb_ch2, emb_use_days, emb_vitality, conv_vit_k, conv_vit_b,
           conv_app_k, conv_app_b, user_W1, user_b1, user_W2, user_b2,
           rate_W1, rate_b1, rate_W2, rate_b2):
    idx_feats = (brand, modelname, version, phone_log_model, phone_raw_model,
                 user_age, user_sex, user_degree, resident_province,
                 resident_city, resident_city_type, sale_channel_1,
                 sale_channel_2)
    tabs = (emb_brand, emb_model, emb_version, emb_phone_log, emb_phone_raw,
            emb_age, emb_sex, emb_degree, emb_province, emb_city,
            emb_city_type, emb_ch1, emb_ch2)
    tab_all, tab_ud = _prep_tables(tabs, emb_use_days)
    offs, acc = [], 0
    for t in tabs:
        offs.append(acc)
        acc += t.shape[0]
    idx_all = jnp.stack(
        [a.reshape(-1).astype(jnp.int32) + o for a, o in zip(idx_feats, offs)],
        axis=1).reshape(-1)
    idx_ud = total_use_days.reshape(-1).astype(jnp.int32)
    out_all, ud = _sc_gather(idx_all, idx_ud, tab_all, tab_ud)
    cat = out_all.reshape(_B, 13 * 128)

    z128 = jnp.zeros((128, 128), _F32)
    z118 = jnp.zeros((118, 128), _F32)
    pw, uw = [], []
    for f in range(5):
        pw += [phone_W1[10 * f:10 * f + 10], z118]
        uw.append(z128)
    for g in range(8):
        pw.append(z128)
        uw += [user_W1[10 * g:10 * g + 10], z118]
    pW1 = jnp.concatenate(pw, axis=0).astype(_BF16)
    uW1c = jnp.concatenate(uw, axis=0).astype(_BF16)
    uW1ud = jnp.concatenate([user_W1[80:180], jnp.zeros((28, 128), _F32)],
                            axis=0).astype(_BF16)
    uWa = jnp.pad(user_W1[180:1044].reshape(27, 32, 128),
                  ((0, 2), (0, 0), (0, 0))).reshape(928, 128).astype(_BF16)
    uWv = user_W1[1044:1908].astype(_BF16)
    M = jnp.einsum('cd,tdo->tco', emb_vitality, conv_vit_k)
    wvb = sum(jnp.einsum('ab,co->cabo', jnp.eye(30, 28, k=-t, dtype=_F32),
                         M[t]) for t in range(3))
    wvb = wvb.reshape(90, 896).astype(_BF16)
    vbB = jnp.tile(conv_vit_b, (28,)).reshape(1, 896)
    kcat = jnp.concatenate([conv_app_k[0], conv_app_k[1], conv_app_k[2]],
                           axis=1).astype(_BF16)

    x2 = all_app.reshape(_B * 30, 609)
    return _tc_towers(
        vatality.astype(jnp.int32), x2, cat, ud,
        kcat, conv_app_b.reshape(1, 32).astype(_BF16), wvb, vbB,
        pW1, phone_b1.reshape(1, 128), phone_W2, phone_b2.reshape(1, 32),
        uW1c, uW1ud, uWa, uWv, user_b1.reshape(1, 128), user_W2,
        user_b2.reshape(1, 32),
        rate_W1, rate_b1.reshape(1, 64), rate_W2, rate_b2.reshape(1, 1))


# speedup vs baseline: 1.5058x; 1.1863x over previous
import functools

import jax
import jax.numpy as jnp
from jax import lax
from jax.experimental import pallas as pl
from jax.experimental.pallas import tpu as pltpu
from jax.experimental.pallas import tpu_sc as plsc

_B = 4096
_BB = 128
_NW = 32
_CHUNK = _B // _NW

_F32 = jnp.float32
_BF16 = jnp.bfloat16


def _sc_gather(idx_all, idx_ud, tab_all, tab_ud):
    mesh = plsc.VectorSubcoreMesh(core_axis_name="c", subcore_axis_name="s")

    @functools.partial(
        pl.kernel,
        mesh=mesh,
        out_type=(jax.ShapeDtypeStruct((_B * 13, 128), _F32),
                  jax.ShapeDtypeStruct((_B, 128), _F32)),
        scratch_types=[
            pltpu.VMEM((13 * _CHUNK,), jnp.int32),
            pltpu.VMEM((_CHUNK,), jnp.int32),
            pltpu.VMEM((_CHUNK, 128), _F32),
            pltpu.VMEM((_CHUNK, 128), _F32),
            pltpu.VMEM((_CHUNK, 128), _F32),
            pltpu.VMEM((_CHUNK, 128), _F32),
            pltpu.SemaphoreType.DMA,
            pltpu.SemaphoreType.DMA,
            pltpu.SemaphoreType.DMA,
            pltpu.SemaphoreType.DMA,
        ],
    )
    def k(idx_all_ref, idx_ud_ref, tab_all_ref, tab_ud_ref, out_ref,
          out_ud_ref, idx_v, idx_ud_v, b0, b1, b2, b3, s0, s1, s2, s3):
        wid = lax.axis_index("s") * 2 + lax.axis_index("c")
        base = wid * (13 * _CHUNK)
        ud_base = wid * _CHUNK
        pltpu.sync_copy(idx_all_ref.at[pl.ds(base, 13 * _CHUNK)], idx_v)
        pltpu.sync_copy(idx_ud_ref.at[pl.ds(ud_base, _CHUNK)], idx_ud_v)
        bufs = (b0, b1, b2, b3)
        sems = (s0, s1, s2, s3)
        cps = [None] * 14

        def fire(j):
            if j < 13:
                cps[j] = pltpu.async_copy(
                    tab_all_ref.at[idx_v.at[pl.ds(_CHUNK * j, _CHUNK)]],
                    bufs[j % 4], sems[j % 4])
            else:
                cps[j] = pltpu.async_copy(tab_ud_ref.at[idx_ud_v],
                                          bufs[j % 4], sems[j % 4])

        def drain(j):
            cps[j].wait()
            if j < 13:
                pltpu.sync_copy(bufs[j % 4],
                                out_ref.at[pl.ds(base + _CHUNK * j, _CHUNK), :])
            else:
                pltpu.sync_copy(bufs[j % 4],
                                out_ud_ref.at[pl.ds(ud_base, _CHUNK), :])

        for j in range(4):
            fire(j)
        for j in range(4, 14):
            drain(j - 4)
            fire(j)
        for j in range(10, 14):
            drain(j)

    return k(idx_all, idx_ud, tab_all, tab_ud)


_TAB_SIZES = (100, 5000, 50, 10000, 10000, 100, 3, 10, 35, 400, 6, 50, 200)


def _prep_body(*refs):
    tab_refs = refs[0:13]
    ud_ref = refs[13]
    out_ref = refs[14]
    out_ud_ref = refs[15]
    off = 0
    for t_ref, v in zip(tab_refs, _TAB_SIZES):
        out_ref[pl.ds(off, v), :] = jnp.concatenate(
            [t_ref[...], jnp.zeros((v, 118), _F32)], axis=1)
        off += v
    out_ud_ref[...] = jnp.concatenate(
        [ud_ref[...], jnp.zeros((5000, 28), _F32)], axis=1)


def _prep_tables(tabs, emb_use_days):
    def full(a):
        nd = a.ndim
        return pl.BlockSpec(a.shape, lambda _n=nd: (0,) * _n)

    return pl.pallas_call(
        _prep_body,
        in_specs=[full(t) for t in tabs] + [full(emb_use_days)],
        out_specs=(pl.BlockSpec((25954, 128), lambda: (0, 0)),
                   pl.BlockSpec((5000, 128), lambda: (0, 0))),
        out_shape=(jax.ShapeDtypeStruct((25954, 128), _F32),
                   jax.ShapeDtypeStruct((5000, 128), _F32)),
    )(*tabs, emb_use_days)


def _tc_body(vat_ref, x2_ref, cat_ref, ud_ref, kcat_ref, appb_ref, wvb_ref,
             vbB_ref, pW1_ref, pb1_ref, pW2_ref, pb2_ref, uW1c_ref,
             uW1ud_ref, uWa_ref, uWv_ref, ub1_ref, uW2_ref, ub2_ref,
             rW1_ref, rb1_ref, rW2_ref, rb2_ref, out_ref):
    _up = lambda a, t: pltpu.roll(a, a.shape[0] - t, 0)

    x = x2_ref[...].reshape(_BB * 30, 609)
    P = jnp.dot(x.astype(_BF16), kcat_ref[...],
                preferred_element_type=_F32)
    y = (P[:, 0:32] + _up(P, 1)[:, 32:64] + _up(P, 2)[:, 64:96]
         + appb_ref[...])

    vat = vat_ref[...]
    oh = jnp.concatenate(
        [jnp.where(vat == c, 1.0, 0.0) for c in range(3)],
        axis=1).astype(_BF16)
    yvB = jnp.dot(oh, wvb_ref[...],
                  preferred_element_type=_F32) + vbB_ref[...]
    yvB = jnp.maximum(yvB, 0.0)
    vit = jnp.maximum(yvB[:, 0:864], yvB[:, 32:896]).astype(_BF16)

    ya3 = jnp.maximum(y, 0.0).reshape(_BB, 30, 32)
    za3 = jnp.maximum(ya3[:, 0:29, :], ya3[:, 1:30, :])
    app = za3.reshape(_BB, 928).astype(_BF16)

    cat = cat_ref[...].astype(_BF16)
    u1 = (jnp.dot(cat, uW1c_ref[...], preferred_element_type=_F32)
          + jnp.dot(ud_ref[...].astype(_BF16), uW1ud_ref[...],
                    preferred_element_type=_F32)
          + jnp.dot(app, uWa_ref[...], preferred_element_type=_F32)
          + jnp.dot(vit, uWv_ref[...], preferred_element_type=_F32)
          + ub1_ref[...])
    u1 = jnp.maximum(u1, 0.0)
    ue = jnp.maximum(
        jnp.dot(u1, uW2_ref[...], preferred_element_type=_F32) + ub2_ref[...],
        0.0)
    p1 = jnp.maximum(
        jnp.dot(cat, pW1_ref[...], preferred_element_type=_F32) + pb1_ref[...],
        0.0)
    pe = jnp.maximum(
        jnp.dot(p1, pW2_ref[...], preferred_element_type=_F32) + pb2_ref[...],
        0.0)
    h = jnp.concatenate([ue, pe], axis=1)
    r1 = jnp.maximum(
        jnp.dot(h, rW1_ref[...], preferred_element_type=_F32) + rb1_ref[...],
        0.0)
    o = jnp.dot(r1, rW2_ref[...], preferred_element_type=_F32) + rb2_ref[...]
    out_ref[...] = jax.nn.sigmoid(o)


def _tc_towers(vat, x2, cat, ud, kcat, appb, wvb, vbB, pW1, pb1, pW2, pb2,
               uW1c, uW1ud, uWa, uWv, ub1, uW2, ub2, rW1, rb1, rW2, rb2):
    def full(a):
        nd = a.ndim
        return pl.BlockSpec(a.shape, lambda i, _n=nd: (0,) * _n)

    weights = (kcat, appb, wvb, vbB, pW1, pb1, pW2, pb2,
               uW1c, uW1ud, uWa, uWv, ub1, uW2, ub2, rW1, rb1, rW2, rb2)
    return pl.pallas_call(
        _tc_body,
        grid=(_B // _BB,),
        in_specs=[
            pl.BlockSpec((_BB, 30), lambda i: (i, 0)),
            pl.BlockSpec((_BB, 30, 609), lambda i: (i, 0, 0)),
            pl.BlockSpec((_BB, 13 * 128), lambda i: (i, 0)),
            pl.BlockSpec((_BB, 128), lambda i: (i, 0)),
        ] + [full(w) for w in weights],
        out_specs=pl.BlockSpec((_BB, 1), lambda i: (i, 0)),
        out_shape=jax.ShapeDtypeStruct((_B, 1), _F32),
    )(vat, x2, cat, ud, *weights)


def kernel(brand, modelname, version, phone_log_model, phone_raw_model,
           total_use_days, user_age, user_sex, user_degree, resident_province,
           resident_city, resident_city_type, sale_channel_1, sale_channel_2,
           vatality, all_app, emb_brand, emb_model, emb_version, emb_phone_log,
           emb_phone_raw, phone_W1, phone_b1, phone_W2, phone_b2, emb_age,
           emb_sex, emb_degree, emb_province, emb_city, emb_city_type, emb_ch1,
           emb_ch2, emb_use_days, emb_vitality, conv_vit_k, conv_vit_b,
           conv_app_k, conv_app_b, user_W1, user_b1, user_W2, user_b2,
           rate_W1, rate_b1, rate_W2, rate_b2):
    idx_feats = (brand, modelname, version, phone_log_model, phone_raw_model,
                 user_age, user_sex, user_degree, resident_province,
                 resident_city, resident_city_type, sale_channel_1,
                 sale_channel_2)
    tabs = (emb_brand, emb_model, emb_version, emb_phone_log, emb_phone_raw,
            emb_age, emb_sex, emb_degree, emb_province, emb_city,
            emb_city_type, emb_ch1, emb_ch2)
    tab_all, tab_ud = _prep_tables(tabs, emb_use_days)
    offs, acc = [], 0
    for t in tabs:
        offs.append(acc)
        acc += t.shape[0]
    idx_all = jnp.stack(
        [a.reshape(-1).astype(jnp.int32) + o for a, o in zip(idx_feats, offs)],
        axis=1).reshape(-1)
    idx_ud = total_use_days.reshape(-1).astype(jnp.int32)
    out_all, ud = _sc_gather(idx_all, idx_ud, tab_all, tab_ud)
    cat = out_all.reshape(_B, 13 * 128)

    z128 = jnp.zeros((128, 128), _F32)
    z118 = jnp.zeros((118, 128), _F32)
    pw, uw = [], []
    for f in range(5):
        pw += [phone_W1[10 * f:10 * f + 10], z118]
        uw.append(z128)
    for g in range(8):
        pw.append(z128)
        uw += [user_W1[10 * g:10 * g + 10], z118]
    pW1 = jnp.concatenate(pw, axis=0).astype(_BF16)
    uW1c = jnp.concatenate(uw, axis=0).astype(_BF16)
    uW1ud = jnp.concatenate([user_W1[80:180], jnp.zeros((28, 128), _F32)],
                            axis=0).astype(_BF16)
    uWa = jnp.pad(user_W1[180:1044].reshape(27, 32, 128),
                  ((0, 2), (0, 0), (0, 0))).reshape(928, 128).astype(_BF16)
    uWv = user_W1[1044:1908].astype(_BF16)
    M = jnp.einsum('cd,tdo->tco', emb_vitality, conv_vit_k)
    wvb = sum(jnp.einsum('ab,co->cabo', jnp.eye(30, 28, k=-t, dtype=_F32),
                         M[t]) for t in range(3))
    wvb = wvb.reshape(90, 896).astype(_BF16)
    vbB = jnp.tile(conv_vit_b, (28,)).reshape(1, 896)
    kcat = jnp.concatenate([conv_app_k[0], conv_app_k[1], conv_app_k[2]],
                           axis=1).astype(_BF16)

    return _tc_towers(
        vatality.astype(jnp.int32), all_app, cat, ud,
        kcat, conv_app_b.reshape(1, 32).astype(_BF16), wvb, vbB,
        pW1, phone_b1.reshape(1, 128), phone_W2, phone_b2.reshape(1, 32),
        uW1c, uW1ud, uWa, uWv, user_b1.reshape(1, 128), user_W2,
        user_b2.reshape(1, 32),
        rate_W1, rate_b1.reshape(1, 64), rate_W2, rate_b2.reshape(1, 1))
